# Initial kernel scaffold; baseline (speedup 1.0000x reference)
import jax
import jax.numpy as jnp
from jax.experimental import pallas as pl


def kernel(inputs, embedding_weight, ema_w, ema_cluster_size):
    raise NotImplementedError("write your pallas kernel here")


# speedup vs baseline: 1.0499x; 1.0499x over previous
import functools

import jax
import jax.numpy as jnp
from jax import lax
from jax.experimental import pallas as pl
from jax.experimental.pallas import tpu as pltpu
from jax.experimental.pallas import tpu_sc as plsc

_N_EMB = 8192
_DIM = 32
_N_TOK = 8192
_BT = 256
_NBLK = _N_TOK // _BT
_DECAY = 0.99
_EPS = 1e-05

_NC = 2
_NS = 16
_NW = _NC * _NS
_BPW = _N_TOK // _NW
_CHUNK = 128


def _k1(enc_ref, flat_ref, emaw_ref, emacs_ref,
        idx_ref, wnew_ref, counts_ref,
        counts_acc, dw_acc):
    i = pl.program_id(0)
    onehot = enc_ref[...]
    flat = flat_ref[...]

    @pl.when(i == 0)
    def _init():
        counts_acc[...] = jnp.zeros_like(counts_acc)
        dw_acc[...] = jnp.zeros_like(dw_acc)

    iota_f = jax.lax.broadcasted_iota(
        jnp.int32, (_BT, _N_EMB), 1).astype(jnp.float32)
    idx_f = jnp.max(onehot * iota_f, axis=1, keepdims=True)
    idx_ref[...] = idx_f.astype(jnp.int32)
    ones = jnp.ones((_BT, 1), jnp.float32)
    counts_acc[...] += jax.lax.dot_general(
        onehot, ones, (((0,), (0,)), ((), ())),
        preferred_element_type=jnp.float32)
    dw_acc[...] += jax.lax.dot_general(
        onehot, flat, (((0,), (0,)), ((), ())),
        preferred_element_type=jnp.float32)

    @pl.when(i == _NBLK - 1)
    def _finish():
        counts = counts_acc[...]
        cs = emacs_ref[...] * _DECAY + (1.0 - _DECAY) * counts
        n = jnp.sum(cs)
        cs = (cs + _EPS) / (n + _N_EMB * _EPS) * n
        ema_w_new = emaw_ref[...] * _DECAY + (1.0 - _DECAY) * dw_acc[...]
        wn = ema_w_new / cs
        wn = wn.astype(jnp.bfloat16).astype(jnp.float32)
        wnew_ref[...] = jnp.concatenate(
            [wn, jnp.zeros((_N_EMB, 128 - _DIM), jnp.float32)], axis=1)
        counts_ref[...] = counts


def _sc_gather(table_hbm, idx_hbm, out_hbm, idx_v, rows_v, sem):
    wid = lax.axis_index("s") * _NC + lax.axis_index("c")
    base = wid * _BPW
    pltpu.sync_copy(idx_hbm.at[wid], idx_v)
    for c in range(_BPW // _CHUNK):
        pltpu.async_copy(
            table_hbm.at[idx_v.at[c]],
            rows_v.at[pl.ds(c * _CHUNK, _CHUNK)], sem).wait()
    pltpu.sync_copy(rows_v, out_hbm.at[pl.ds(base, _BPW)])


def _epilogue(flat_ref, q_ref, counts_ref, qst_ref, loss_ref, perp_ref):
    flat = flat_ref[...]
    diff = q_ref[:, : _DIM] - flat
    qst_ref[...] = flat + diff
    loss_ref[...] = jnp.full(
        (1, 1), jnp.sum(diff * diff) / jnp.float32(_N_TOK * _DIM))
    ap = counts_ref[...] * jnp.float32(1.0 / _N_TOK)
    ent = jnp.sum(ap * jnp.log(ap + 1e-10))
    perp_ref[...] = jnp.full((1, 1), jnp.exp(-ent))


def kernel(inputs, embedding_weight, ema_w, ema_cluster_size):
    input_shape = inputs.shape
    flat_input = inputs.reshape(-1, _DIM)
    distances = (jnp.sum(flat_input ** 2, axis=1, keepdims=True)
                 + jnp.sum(embedding_weight ** 2, axis=1)
                 - 2.0 * jnp.matmul(flat_input, embedding_weight.T))
    encoding_indices = jnp.argmin(distances, axis=1)
    encodings = jnp.zeros((_N_TOK, _N_EMB), dtype=jnp.float32).at[
        jnp.arange(_N_TOK), encoding_indices].set(1.0)

    emacs = ema_cluster_size.reshape(-1, 1)

    idx, wnew, counts = pl.pallas_call(
        _k1,
        grid=(_NBLK,),
        in_specs=[
            pl.BlockSpec((_BT, _N_EMB), lambda i: (i, 0)),
            pl.BlockSpec((_BT, _DIM), lambda i: (i, 0)),
            pl.BlockSpec((_N_EMB, _DIM), lambda i: (0, 0)),
            pl.BlockSpec((_N_EMB, 1), lambda i: (0, 0)),
        ],
        out_specs=[
            pl.BlockSpec((_BT, 1), lambda i: (i, 0)),
            pl.BlockSpec((_N_EMB, 128), lambda i: (0, 0)),
            pl.BlockSpec((_N_EMB, 1), lambda i: (0, 0)),
        ],
        out_shape=[
            jax.ShapeDtypeStruct((_N_TOK, 1), jnp.int32),
            jax.ShapeDtypeStruct((_N_EMB, 128), jnp.float32),
            jax.ShapeDtypeStruct((_N_EMB, 1), jnp.float32),
        ],
        scratch_shapes=[
            pltpu.VMEM((_N_EMB, 1), jnp.float32),
            pltpu.VMEM((_N_EMB, _DIM), jnp.float32),
        ],
    )(encodings, flat_input, ema_w, emacs)

    idx_w = idx.reshape(_NW, _BPW // _CHUNK, _CHUNK)

    sc_gather = functools.partial(
        pl.kernel,
        out_type=jax.ShapeDtypeStruct((_N_TOK, 128), jnp.float32),
        mesh=plsc.VectorSubcoreMesh(core_axis_name="c", subcore_axis_name="s"),
        scratch_types=[
            pltpu.VMEM((_BPW // _CHUNK, _CHUNK), jnp.int32),
            pltpu.VMEM((_BPW, 128), jnp.float32),
            pltpu.SemaphoreType.DMA,
        ],
    )(_sc_gather)
    q = sc_gather(wnew, idx_w)

    qst, loss, perp = pl.pallas_call(
        _epilogue,
        in_specs=[
            pl.BlockSpec((_N_TOK, _DIM), lambda: (0, 0)),
            pl.BlockSpec((_N_TOK, 128), lambda: (0, 0)),
            pl.BlockSpec((_N_EMB, 1), lambda: (0, 0)),
        ],
        out_specs=[
            pl.BlockSpec((_N_TOK, _DIM), lambda: (0, 0)),
            pl.BlockSpec((1, 1), lambda: (0, 0)),
            pl.BlockSpec((1, 1), lambda: (0, 0)),
        ],
        out_shape=[
            jax.ShapeDtypeStruct((_N_TOK, _DIM), jnp.float32),
            jax.ShapeDtypeStruct((1, 1), jnp.float32),
            jax.ShapeDtypeStruct((1, 1), jnp.float32),
        ],
    )(flat_input, q, counts)

    return (qst.reshape(input_shape), encodings, loss[0, 0], perp[0, 0])
